# baseline (speedup 1.0000x reference)
import functools

import jax
import jax.numpy as jnp
from jax import lax
from jax.experimental import pallas as pl
from jax.experimental.pallas import tpu as pltpu
from jax.experimental.pallas import tpu_sc as plsc

D_MODEL = 256
ANGLE_K = 50
TEMPERATURE = 10000.0
SCALE = 1.0
NUM_POS_FEATS = (D_MODEL // 3) // 2 * 2
PADDING = D_MODEL - NUM_POS_FEATS * 3

N = 16384
M = 2048
K = ANGLE_K

NBA = 512
NBB = 1024
MBE = 256
NEG = -3.0e38


def _a_body(pts_ref, nodes_ref, p2n_ref, d2_ref, cnt_ref):
    px = pts_ref[0:1, :]
    py = pts_ref[1:2, :]
    pz = pts_ref[2:3, :]
    nx = nodes_ref[:, 0:1]
    ny = nodes_ref[:, 1:2]
    nz = nodes_ref[:, 2:3]
    dx = nx - px
    dy = ny - py
    dz = nz - pz
    sq = dx * dx + dy * dy + dz * dz
    mn = jnp.min(sq, axis=0, keepdims=True)
    nid = lax.broadcasted_iota(jnp.int32, (M, NBA), 0)
    amn = jnp.min(jnp.where(sq == mn, nid, jnp.int32(M)), axis=0)
    p2n_ref[0, 0, :] = amn
    d2_ref[0, 0, :] = mn[0]
    onehot = amn[None, :] == lax.broadcasted_iota(jnp.int32, (M, NBA), 0)
    cnt = jnp.sum(onehot.astype(jnp.int32), axis=1, keepdims=True)

    @pl.when(pl.program_id(0) == 0)
    def _():
        cnt_ref[...] = jnp.zeros((M, 1), jnp.int32)

    cnt_ref[...] += cnt


def _run_a(points_t, nodes):
    return pl.pallas_call(
        _a_body,
        grid=(N // NBA,),
        in_specs=[
            pl.BlockSpec((3, NBA), lambda i: (0, i)),
            pl.BlockSpec((M, 3), lambda i: (0, 0)),
        ],
        out_specs=[
            pl.BlockSpec((1, 1, NBA), lambda i: (i, 0, 0)),
            pl.BlockSpec((1, 1, NBA), lambda i: (i, 0, 0)),
            pl.BlockSpec((M, 1), lambda i: (0, 0)),
        ],
        out_shape=[
            jax.ShapeDtypeStruct((N // NBA, 1, NBA), jnp.int32),
            jax.ShapeDtypeStruct((N // NBA, 1, NBA), jnp.float32),
            jax.ShapeDtypeStruct((M, 1), jnp.int32),
        ],
    )(points_t, nodes)


NBLK = N // NBB
_TRI_STARTS = [t * (2 * NBLK + 1 - t) // 2 for t in range(NBLK + 1)]


def _tri_a(g):
    a = jnp.int32(0)
    for t in range(1, NBLK):
        a = a + (g >= _TRI_STARTS[t]).astype(jnp.int32)
    return a


def _tri_ab(g):
    a = _tri_a(g)
    b = g - a * (2 * NBLK + 1 - a) // 2 + a
    return a, b


def _b_body(p2nj_ref, d2j_ref, p2nk_ref, d2k_ref, rankj_ref, rankk_ref,
            kacc_ref):
    g = pl.program_id(0)
    a, b = _tri_ab(g)
    diag = a == b
    pj = p2nj_ref[0]
    dj = d2j_ref[0]
    pk = p2nk_ref[0, 0, :, :]
    dk = d2k_ref[0, 0, :, :]
    same = pk == pj
    ltm = dk < dj
    fsame = jnp.where(same, 1.0, 0.0)
    fsl = jnp.where(same & ltm, 1.0, 0.0)
    ones_r = jnp.ones((8, NBB), jnp.float32)
    ones_c = jnp.ones((NBB, 8), jnp.float32)

    @pl.when(g == 0)
    def _():
        kacc_ref[...] = jnp.zeros((NBLK, NBB, 1), jnp.int32)

    @pl.when(diag)
    def _():
        jl = lax.broadcasted_iota(jnp.int32, (NBB, NBB), 1)
        kl = lax.broadcasted_iota(jnp.int32, (NBB, NBB), 0)
        fslt = jnp.where(same & (ltm | ((dk == dj) & (kl < jl))), 1.0, 0.0)
        cj = jnp.dot(ones_r, fslt, preferred_element_type=jnp.float32)
        rankj_ref[0, 0, :] = cj[0].astype(jnp.int32)
        rankk_ref[0, :, :] = kacc_ref[a]

    @pl.when(jnp.logical_not(diag))
    def _():
        cj = jnp.dot(ones_r, fsl, preferred_element_type=jnp.float32)
        rankj_ref[0, 0, :] += cj[0].astype(jnp.int32)
        nsame = jnp.dot(fsame, ones_c, preferred_element_type=jnp.float32)
        nsl = jnp.dot(fsl, ones_c, preferred_element_type=jnp.float32)
        kacc_ref[b] += (nsame[:, 0:1] - nsl[:, 0:1]).astype(jnp.int32)


def _run_b(p2n_row, d2_row, p2n_col, d2_col):
    ntri = NBLK * (NBLK + 1) // 2

    def ja(g):
        a, _ = _tri_ab(g)
        return (a, 0, 0)

    def jb3(g):
        _, b = _tri_ab(g)
        return (b, 0, 0)

    def kb4(g):
        _, b = _tri_ab(g)
        return (b, 0, 0, 0)

    return pl.pallas_call(
        _b_body,
        grid=(ntri,),
        in_specs=[
            pl.BlockSpec((1, 1, NBB), ja),
            pl.BlockSpec((1, 1, NBB), ja),
            pl.BlockSpec((1, 1, NBB, 1), kb4),
            pl.BlockSpec((1, 1, NBB, 1), kb4),
        ],
        out_specs=[
            pl.BlockSpec((1, 1, NBB), ja),
            pl.BlockSpec((1, NBB, 1), ja),
        ],
        out_shape=[
            jax.ShapeDtypeStruct((NBLK, 1, NBB), jnp.int32),
            jax.ShapeDtypeStruct((NBLK, NBB, 1), jnp.int32),
        ],
        scratch_shapes=[pltpu.VMEM((NBLK, NBB, 1), jnp.int32)],
    )(p2n_row, d2_row, p2n_col, d2_col)


def _run_c(payload_t, p2n, rank):
    info = plsc.get_sparse_core_info()
    nw = info.num_cores * info.num_subcores
    mpt = M // nw
    ch = 2048
    nst = N // ch
    mesh = plsc.VectorSubcoreMesh(core_axis_name="c", subcore_axis_name="s")

    @functools.partial(
        pl.kernel,
        mesh=mesh,
        compiler_params=pltpu.CompilerParams(needs_layout_passes=False),
        out_type=jax.ShapeDtypeStruct((M // 64, 6 * K * 64), jnp.float32),
        scratch_types=[
            pltpu.VMEM((8, ch), jnp.float32),
            pltpu.VMEM((ch,), jnp.int32),
            pltpu.VMEM((ch,), jnp.int32),
            pltpu.VMEM((6 * K * mpt,), jnp.float32),
        ],
    )
    def c_kernel(payload_hbm, p2n_hbm, rank_hbm, out_hbm, buf, p2n_v, rank_v,
                 tab):
        wid = lax.axis_index("s") * info.num_cores + lax.axis_index("c")
        base_node = wid * mpt

        def stage(st, _):
            pltpu.sync_copy(p2n_hbm.at[pl.ds(st * ch, ch)], p2n_v)
            pltpu.sync_copy(rank_hbm.at[pl.ds(st * ch, ch)], rank_v)
            pltpu.sync_copy(payload_hbm.at[:, pl.ds(st * ch, ch)], buf)

            def chunk(t, _):
                off = t * 16
                p16 = p2n_v[pl.ds(off, 16)]
                r16 = rank_v[pl.ds(off, 16)]
                loc = p16 - base_node
                msk = (loc >= 0) & (loc < mpt) & (r16 < K)
                rowbase = r16 * mpt + loc
                for c in range(6):
                    v16 = buf[c, pl.ds(off, 16)]
                    plsc.store_scatter(tab, [c * (K * mpt) + rowbase], v16,
                                       mask=msk)
                return 0

            lax.fori_loop(0, ch // 16, chunk, 0)
            return 0

        lax.fori_loop(0, nst, stage, 0)
        pltpu.sync_copy(tab, out_hbm.at[wid])

    return c_kernel(payload_t, p2n, rank)


def _signed_angle_cols(ijx, ijy, ijz, nx, ny, nz):
    cx = ijy * nz - ijz * ny
    cy = ijz * nx - ijx * nz
    cz = ijx * ny - ijy * nx
    sin_v = jnp.sqrt(cx * cx + cy * cy + cz * cz)
    cos_v = ijx * nx + ijy * ny + ijz * nz
    ang = lax.atan2(sin_v, cos_v)
    return jnp.where(cos_v < 0.0, -ang, ang)


def _e_body(s_ref, nodes_ref, nn_ref, cnt_ref, hd_ref, hp2n_ref, wb_ref,
            globt_ref, loct_ref):
    i = pl.program_id(0)
    nx = nodes_ref[0:1, :]
    ny = nodes_ref[1:2, :]
    nz = nodes_ref[2:3, :]
    n1x = nn_ref[0:1, :]
    n1y = nn_ref[1:2, :]
    n1z = nn_ref[2:3, :]
    c_row = cnt_ref[...]

    def feats(px, py, pz, knx, kny, knz):
        ijx = px - nx
        ijy = py - ny
        ijz = pz - nz
        d_ind = jnp.sqrt(ijx * ijx + ijy * ijy + ijz * ijz) * SCALE
        a1 = _signed_angle_cols(ijx, ijy, ijz, n1x, n1y, n1z)
        a2 = _signed_angle_cols(-ijx, -ijy, -ijz, knx, kny, knz)
        a3 = _signed_angle_cols(n1x + 0.0 * px, n1y + 0.0 * px,
                                n1z + 0.0 * px, knx, kny, knz)
        return d_ind, a1, a2, a3

    d_a, a1_a, a2_a, a3_a = feats(s_ref[0], s_ref[1], s_ref[2],
                                  s_ref[3], s_ref[4], s_ref[5])
    valid_a = lax.broadcasted_iota(jnp.int32, (K, MBE), 0) < c_row

    px_f = hd_ref[:, 0:1]
    py_f = hd_ref[:, 1:2]
    pz_f = hd_ref[:, 2:3]
    knx_f = hd_ref[:, 3:4]
    kny_f = hd_ref[:, 4:5]
    knz_f = hd_ref[:, 5:6]
    hp2n = hp2n_ref[...]
    ids_row = i * MBE + lax.broadcasted_iota(jnp.int32, (1, MBE), 1)
    m = hp2n != ids_row
    jr = lax.broadcasted_iota(jnp.int32, (64, 64), 0)
    jc = lax.broadcasted_iota(jnp.int32, (64, 64), 1)
    ltri = (jc < jr).astype(jnp.float32)
    poscnt = jnp.dot(ltri, m.astype(jnp.float32),
                     preferred_element_type=jnp.float32)
    navail = (K - c_row).astype(jnp.float32)
    fmask = m & (poscnt < navail)
    d_f, a1_f, a2_f, a3_f = feats(px_f, py_f, pz_f, knx_f, kny_f, knz_f)

    def one_d(d):
        w0 = wb_ref[d, 0]
        w1 = wb_ref[d, 1]
        w2 = wb_ref[d, 2]
        w3 = wb_ref[d, 3]
        bd = wb_ref[d, 4]
        pa = d_a * w0 + a1_a * w1 + a2_a * w2 + a3_a * w3 + bd
        pf = d_f * w0 + a1_f * w1 + a2_f * w2 + a3_f * w3 + bd
        ra = jnp.max(jnp.where(valid_a, pa, NEG), axis=0, keepdims=True)
        rf = jnp.max(jnp.where(fmask, pf, NEG), axis=0, keepdims=True)
        globt_ref[pl.ds(d, 1), :] = jnp.maximum(ra, rf)

    def dstep(h, _):
        one_d(2 * h)
        one_d(2 * h + 1)
        return 0

    lax.fori_loop(0, D_MODEL // 2, dstep, 0)

    rr = lax.broadcasted_iota(jnp.int32, (D_MODEL, 1), 0)
    dsel = rr // NUM_POS_FEATS
    within = rr - dsel * NUM_POS_FEATS
    ef = (2.0 * jnp.floor(within.astype(jnp.float32) / 2.0)) / float(NUM_POS_FEATS)
    dim_t = jnp.power(jnp.float32(TEMPERATURE), ef)
    xsel = jnp.where(dsel == 0, nx, jnp.where(dsel == 1, ny, nz)) * SCALE
    ph = xsel / dim_t
    val = jnp.where(rr % 2 == 0, jnp.sin(ph), jnp.cos(ph))
    loct_ref[...] = jnp.where(rr < 3 * NUM_POS_FEATS, val,
                              jnp.zeros((D_MODEL, MBE), jnp.float32))


def _run_e(s_tab, nodes_t, nn_t, counts_row, hd, hp2n, wb):
    return pl.pallas_call(
        _e_body,
        grid=(M // MBE,),
        in_specs=[
            pl.BlockSpec((6, K, MBE), lambda i: (0, 0, i)),
            pl.BlockSpec((3, MBE), lambda i: (0, i)),
            pl.BlockSpec((3, MBE), lambda i: (0, i)),
            pl.BlockSpec((1, MBE), lambda i: (0, i)),
            pl.BlockSpec((64, 8), lambda i: (0, 0)),
            pl.BlockSpec((64, 1), lambda i: (0, 0)),
            pl.BlockSpec(memory_space=pltpu.SMEM),
        ],
        out_specs=[
            pl.BlockSpec((D_MODEL, MBE), lambda i: (0, i)),
            pl.BlockSpec((D_MODEL, MBE), lambda i: (0, i)),
        ],
        out_shape=[
            jax.ShapeDtypeStruct((D_MODEL, M), jnp.float32),
            jax.ShapeDtypeStruct((D_MODEL, M), jnp.float32),
        ],
    )(s_tab, nodes_t, nn_t, counts_row, hd, hp2n, wb)


def kernel(points, nodes, points_normals, nodes_normals, glo_proj_W,
           glo_proj_b):
    pts = points[0]
    nds = nodes[0]
    pn = points_normals[0]
    nn = nodes_normals[0]

    p2n3, d23, counts2 = _run_a(pts.T, nds)
    p2n = p2n3.reshape(N)
    d2 = d23.reshape(N)
    counts_row = counts2.reshape(1, M)

    nbb = N // NBB
    rj, rk = _run_b(p2n.reshape(nbb, 1, NBB), d2.reshape(nbb, 1, NBB),
                    p2n.reshape(nbb, 1, NBB, 1), d2.reshape(nbb, 1, NBB, 1))
    rank = rj.reshape(N) + rk.reshape(N)

    payload_t = jnp.concatenate(
        [pts.T, pn.T, jnp.zeros((2, N), jnp.float32)], axis=0)
    s_raw = _run_c(payload_t, p2n, rank).reshape(M // 64, 6, K, 64)
    s_tab = jnp.transpose(s_raw, (1, 2, 0, 3)).reshape(6, K, M)

    hd = jnp.concatenate(
        [jnp.concatenate([pts[:K], pn[:K]], axis=1),
         jnp.zeros((64 - K, 6), jnp.float32)], axis=0)
    hd = jnp.concatenate([hd, jnp.zeros((64, 2), jnp.float32)], axis=1)
    hp2n = jnp.concatenate(
        [p2n[:K], jnp.full((64 - K,), -1, jnp.int32)], axis=0).reshape(64, 1)
    wb = jnp.concatenate(
        [glo_proj_W, glo_proj_b[:, None],
         jnp.zeros((D_MODEL, 3), jnp.float32)], axis=1)

    globt, loct = _run_e(s_tab, nds.T, nn.T, counts_row, hd, hp2n, wb)
    return globt.T[None], loct.T[None]


# speedup vs baseline: 1.0065x; 1.0065x over previous
import functools

import jax
import jax.numpy as jnp
from jax import lax
from jax.experimental import pallas as pl
from jax.experimental.pallas import tpu as pltpu
from jax.experimental.pallas import tpu_sc as plsc

D_MODEL = 256
ANGLE_K = 50
TEMPERATURE = 10000.0
SCALE = 1.0
NUM_POS_FEATS = (D_MODEL // 3) // 2 * 2
PADDING = D_MODEL - NUM_POS_FEATS * 3

N = 16384
M = 2048
K = ANGLE_K

NBA = 512
NBB = 1024
MBE = 512
NEG = -3.0e38


def _a_body(pts_ref, nodes_ref, p2n_ref, d2_ref, cnt_ref):
    px = pts_ref[0:1, :]
    py = pts_ref[1:2, :]
    pz = pts_ref[2:3, :]
    nx = nodes_ref[:, 0:1]
    ny = nodes_ref[:, 1:2]
    nz = nodes_ref[:, 2:3]
    dx = nx - px
    dy = ny - py
    dz = nz - pz
    sq = dx * dx + dy * dy + dz * dz
    mn = jnp.min(sq, axis=0, keepdims=True)
    nid = lax.broadcasted_iota(jnp.int32, (M, NBA), 0)
    amn = jnp.min(jnp.where(sq == mn, nid, jnp.int32(M)), axis=0)
    p2n_ref[0, 0, :] = amn
    d2_ref[0, 0, :] = mn[0]
    onehot = amn[None, :] == lax.broadcasted_iota(jnp.int32, (M, NBA), 0)
    fone = jnp.where(onehot, 1.0, 0.0)
    cntf = jnp.dot(fone, jnp.ones((NBA, 8), jnp.float32),
                   preferred_element_type=jnp.float32)
    cnt = cntf[:, 0:1].astype(jnp.int32)

    @pl.when(pl.program_id(0) == 0)
    def _():
        cnt_ref[...] = jnp.zeros((M, 1), jnp.int32)

    cnt_ref[...] += cnt


def _run_a(points_t, nodes):
    return pl.pallas_call(
        _a_body,
        grid=(N // NBA,),
        in_specs=[
            pl.BlockSpec((3, NBA), lambda i: (0, i)),
            pl.BlockSpec((M, 3), lambda i: (0, 0)),
        ],
        out_specs=[
            pl.BlockSpec((1, 1, NBA), lambda i: (i, 0, 0)),
            pl.BlockSpec((1, 1, NBA), lambda i: (i, 0, 0)),
            pl.BlockSpec((M, 1), lambda i: (0, 0)),
        ],
        out_shape=[
            jax.ShapeDtypeStruct((N // NBA, 1, NBA), jnp.int32),
            jax.ShapeDtypeStruct((N // NBA, 1, NBA), jnp.float32),
            jax.ShapeDtypeStruct((M, 1), jnp.int32),
        ],
    )(points_t, nodes)


NBLK = N // NBB
_TRI_STARTS = [t * (2 * NBLK + 1 - t) // 2 for t in range(NBLK + 1)]


def _tri_a(g):
    a = jnp.int32(0)
    for t in range(1, NBLK):
        a = a + (g >= _TRI_STARTS[t]).astype(jnp.int32)
    return a


def _tri_ab(g):
    a = _tri_a(g)
    b = g - a * (2 * NBLK + 1 - a) // 2 + a
    return a, b


def _b_body(p2nj_ref, d2j_ref, p2nk_ref, d2k_ref, rankj_ref, rankk_ref,
            kacc_ref):
    g = pl.program_id(0)
    a, b = _tri_ab(g)
    diag = a == b
    pj = p2nj_ref[0]
    dj = d2j_ref[0]
    pk = p2nk_ref[0, 0, :, :]
    dk = d2k_ref[0, 0, :, :]
    same = pk == pj
    ltm = dk < dj
    fsame = jnp.where(same, 1.0, 0.0)
    fsl = jnp.where(same & ltm, 1.0, 0.0)
    ones_r = jnp.ones((8, NBB), jnp.float32)
    ones_c = jnp.ones((NBB, 8), jnp.float32)

    @pl.when(g == 0)
    def _():
        kacc_ref[...] = jnp.zeros((NBLK, NBB, 1), jnp.int32)

    @pl.when(diag)
    def _():
        jl = lax.broadcasted_iota(jnp.int32, (NBB, NBB), 1)
        kl = lax.broadcasted_iota(jnp.int32, (NBB, NBB), 0)
        fslt = jnp.where(same & (ltm | ((dk == dj) & (kl < jl))), 1.0, 0.0)
        cj = jnp.dot(ones_r, fslt, preferred_element_type=jnp.float32)
        rankj_ref[0, 0, :] = cj[0].astype(jnp.int32)
        rankk_ref[0, :, :] = kacc_ref[a]

    @pl.when(jnp.logical_not(diag))
    def _():
        cj = jnp.dot(ones_r, fsl, preferred_element_type=jnp.float32)
        rankj_ref[0, 0, :] += cj[0].astype(jnp.int32)
        nsame = jnp.dot(fsame, ones_c, preferred_element_type=jnp.float32)
        nsl = jnp.dot(fsl, ones_c, preferred_element_type=jnp.float32)
        kacc_ref[b] += (nsame[:, 0:1] - nsl[:, 0:1]).astype(jnp.int32)


def _run_b(p2n_row, d2_row, p2n_col, d2_col):
    ntri = NBLK * (NBLK + 1) // 2

    def ja(g):
        a, _ = _tri_ab(g)
        return (a, 0, 0)

    def jb3(g):
        _, b = _tri_ab(g)
        return (b, 0, 0)

    def kb4(g):
        _, b = _tri_ab(g)
        return (b, 0, 0, 0)

    return pl.pallas_call(
        _b_body,
        grid=(ntri,),
        in_specs=[
            pl.BlockSpec((1, 1, NBB), ja),
            pl.BlockSpec((1, 1, NBB), ja),
            pl.BlockSpec((1, 1, NBB, 1), kb4),
            pl.BlockSpec((1, 1, NBB, 1), kb4),
        ],
        out_specs=[
            pl.BlockSpec((1, 1, NBB), ja),
            pl.BlockSpec((1, NBB, 1), ja),
        ],
        out_shape=[
            jax.ShapeDtypeStruct((NBLK, 1, NBB), jnp.int32),
            jax.ShapeDtypeStruct((NBLK, NBB, 1), jnp.int32),
        ],
        scratch_shapes=[pltpu.VMEM((NBLK, NBB, 1), jnp.int32)],
    )(p2n_row, d2_row, p2n_col, d2_col)


def _run_c(payload_t, p2n, rank):
    info = plsc.get_sparse_core_info()
    nw = info.num_cores * info.num_subcores
    mpt = M // nw
    ch = 2048
    nst = N // ch
    mesh = plsc.VectorSubcoreMesh(core_axis_name="c", subcore_axis_name="s")

    @functools.partial(
        pl.kernel,
        mesh=mesh,
        compiler_params=pltpu.CompilerParams(needs_layout_passes=False),
        out_type=jax.ShapeDtypeStruct((M // 64, 6 * K * 64), jnp.float32),
        scratch_types=[
            pltpu.VMEM((8, ch), jnp.float32),
            pltpu.VMEM((ch,), jnp.int32),
            pltpu.VMEM((ch,), jnp.int32),
            pltpu.VMEM((6 * K * mpt,), jnp.float32),
        ],
    )
    def c_kernel(payload_hbm, p2n_hbm, rank_hbm, out_hbm, buf, p2n_v, rank_v,
                 tab):
        wid = lax.axis_index("s") * info.num_cores + lax.axis_index("c")
        base_node = wid * mpt

        def stage(st, _):
            pltpu.sync_copy(p2n_hbm.at[pl.ds(st * ch, ch)], p2n_v)
            pltpu.sync_copy(rank_hbm.at[pl.ds(st * ch, ch)], rank_v)
            pltpu.sync_copy(payload_hbm.at[:, pl.ds(st * ch, ch)], buf)

            def chunk(t, _):
                off = t * 16
                p16 = p2n_v[pl.ds(off, 16)]
                r16 = rank_v[pl.ds(off, 16)]
                loc = p16 - base_node
                msk = (loc >= 0) & (loc < mpt) & (r16 < K)
                rowbase = r16 * mpt + loc
                for c in range(6):
                    v16 = buf[c, pl.ds(off, 16)]
                    plsc.store_scatter(tab, [c * (K * mpt) + rowbase], v16,
                                       mask=msk)
                return 0

            lax.fori_loop(0, ch // 16, chunk, 0)
            return 0

        lax.fori_loop(0, nst, stage, 0)
        pltpu.sync_copy(tab, out_hbm.at[wid])

    return c_kernel(payload_t, p2n, rank)


def _signed_angle_cols(ijx, ijy, ijz, nx, ny, nz):
    cx = ijy * nz - ijz * ny
    cy = ijz * nx - ijx * nz
    cz = ijx * ny - ijy * nx
    sin_v = jnp.sqrt(cx * cx + cy * cy + cz * cz)
    cos_v = ijx * nx + ijy * ny + ijz * nz
    ang = lax.atan2(sin_v, cos_v)
    return jnp.where(cos_v < 0.0, -ang, ang)


def _e_body(s_ref, nodes_ref, nn_ref, cnt_ref, hd_ref, hp2n_ref, wb_ref,
            globt_ref, loct_ref):
    i = pl.program_id(0)
    nx = nodes_ref[0:1, :]
    ny = nodes_ref[1:2, :]
    nz = nodes_ref[2:3, :]
    n1x = nn_ref[0:1, :]
    n1y = nn_ref[1:2, :]
    n1z = nn_ref[2:3, :]
    c_row = cnt_ref[...]

    def feats(px, py, pz, knx, kny, knz):
        ijx = px - nx
        ijy = py - ny
        ijz = pz - nz
        d_ind = jnp.sqrt(ijx * ijx + ijy * ijy + ijz * ijz) * SCALE
        a1 = _signed_angle_cols(ijx, ijy, ijz, n1x, n1y, n1z)
        a2 = _signed_angle_cols(-ijx, -ijy, -ijz, knx, kny, knz)
        a3 = _signed_angle_cols(n1x + 0.0 * px, n1y + 0.0 * px,
                                n1z + 0.0 * px, knx, kny, knz)
        return d_ind, a1, a2, a3

    d_a, a1_a, a2_a, a3_a = feats(s_ref[0], s_ref[1], s_ref[2],
                                  s_ref[3], s_ref[4], s_ref[5])
    valid_a = lax.broadcasted_iota(jnp.int32, (K, MBE), 0) < c_row

    px_f = hd_ref[:, 0:1]
    py_f = hd_ref[:, 1:2]
    pz_f = hd_ref[:, 2:3]
    knx_f = hd_ref[:, 3:4]
    kny_f = hd_ref[:, 4:5]
    knz_f = hd_ref[:, 5:6]
    hp2n = hp2n_ref[...]
    ids_row = i * MBE + lax.broadcasted_iota(jnp.int32, (1, MBE), 1)
    m = hp2n != ids_row
    jr = lax.broadcasted_iota(jnp.int32, (64, 64), 0)
    jc = lax.broadcasted_iota(jnp.int32, (64, 64), 1)
    ltri = (jc < jr).astype(jnp.float32)
    poscnt = jnp.dot(ltri, m.astype(jnp.float32),
                     preferred_element_type=jnp.float32)
    navail = (K - c_row).astype(jnp.float32)
    fmask = m & (poscnt < navail)
    d_f, a1_f, a2_f, a3_f = feats(px_f, py_f, pz_f, knx_f, kny_f, knz_f)

    def one_d(d):
        w0 = wb_ref[d, 0]
        w1 = wb_ref[d, 1]
        w2 = wb_ref[d, 2]
        w3 = wb_ref[d, 3]
        bd = wb_ref[d, 4]
        pa = d_a * w0 + a1_a * w1 + a2_a * w2 + a3_a * w3 + bd
        pf = d_f * w0 + a1_f * w1 + a2_f * w2 + a3_f * w3 + bd
        ra = jnp.max(jnp.where(valid_a, pa, NEG), axis=0, keepdims=True)
        rf = jnp.max(jnp.where(fmask, pf, NEG), axis=0, keepdims=True)
        globt_ref[pl.ds(d, 1), :] = jnp.maximum(ra, rf)

    def dstep(h, _):
        one_d(2 * h)
        one_d(2 * h + 1)
        return 0

    lax.fori_loop(0, D_MODEL // 2, dstep, 0)

    rr = lax.broadcasted_iota(jnp.int32, (D_MODEL, 1), 0)
    dsel = rr // NUM_POS_FEATS
    within = rr - dsel * NUM_POS_FEATS
    ef = (2.0 * jnp.floor(within.astype(jnp.float32) / 2.0)) / float(NUM_POS_FEATS)
    dim_t = jnp.power(jnp.float32(TEMPERATURE), ef)
    xsel = jnp.where(dsel == 0, nx, jnp.where(dsel == 1, ny, nz)) * SCALE
    ph = xsel / dim_t
    val = jnp.where(rr % 2 == 0, jnp.sin(ph), jnp.cos(ph))
    loct_ref[...] = jnp.where(rr < 3 * NUM_POS_FEATS, val,
                              jnp.zeros((D_MODEL, MBE), jnp.float32))


def _run_e(s_tab, nodes_t, nn_t, counts_row, hd, hp2n, wb):
    return pl.pallas_call(
        _e_body,
        grid=(M // MBE,),
        in_specs=[
            pl.BlockSpec((6, K, MBE), lambda i: (0, 0, i)),
            pl.BlockSpec((3, MBE), lambda i: (0, i)),
            pl.BlockSpec((3, MBE), lambda i: (0, i)),
            pl.BlockSpec((1, MBE), lambda i: (0, i)),
            pl.BlockSpec((64, 8), lambda i: (0, 0)),
            pl.BlockSpec((64, 1), lambda i: (0, 0)),
            pl.BlockSpec(memory_space=pltpu.SMEM),
        ],
        out_specs=[
            pl.BlockSpec((D_MODEL, MBE), lambda i: (0, i)),
            pl.BlockSpec((D_MODEL, MBE), lambda i: (0, i)),
        ],
        out_shape=[
            jax.ShapeDtypeStruct((D_MODEL, M), jnp.float32),
            jax.ShapeDtypeStruct((D_MODEL, M), jnp.float32),
        ],
    )(s_tab, nodes_t, nn_t, counts_row, hd, hp2n, wb)


def kernel(points, nodes, points_normals, nodes_normals, glo_proj_W,
           glo_proj_b):
    pts = points[0]
    nds = nodes[0]
    pn = points_normals[0]
    nn = nodes_normals[0]

    p2n3, d23, counts2 = _run_a(pts.T, nds)
    p2n = p2n3.reshape(N)
    d2 = d23.reshape(N)
    counts_row = counts2.reshape(1, M)

    nbb = N // NBB
    rj, rk = _run_b(p2n.reshape(nbb, 1, NBB), d2.reshape(nbb, 1, NBB),
                    p2n.reshape(nbb, 1, NBB, 1), d2.reshape(nbb, 1, NBB, 1))
    rank = rj.reshape(N) + rk.reshape(N)

    payload_t = jnp.concatenate(
        [pts.T, pn.T, jnp.zeros((2, N), jnp.float32)], axis=0)
    s_raw = _run_c(payload_t, p2n, rank).reshape(M // 64, 6, K, 64)
    s_tab = jnp.transpose(s_raw, (1, 2, 0, 3)).reshape(6, K, M)

    hd = jnp.concatenate(
        [jnp.concatenate([pts[:K], pn[:K]], axis=1),
         jnp.zeros((64 - K, 6), jnp.float32)], axis=0)
    hd = jnp.concatenate([hd, jnp.zeros((64, 2), jnp.float32)], axis=1)
    hp2n = jnp.concatenate(
        [p2n[:K], jnp.full((64 - K,), -1, jnp.int32)], axis=0).reshape(64, 1)
    wb = jnp.concatenate(
        [glo_proj_W, glo_proj_b[:, None],
         jnp.zeros((D_MODEL, 3), jnp.float32)], axis=1)

    globt, loct = _run_e(s_tab, nds.T, nn.T, counts_row, hd, hp2n, wb)
    return globt.T[None], loct.T[None]
